# baseline (speedup 1.0000x reference)
import functools

import numpy as np
import jax
import jax.numpy as jnp
from jax import lax
from jax.experimental import pallas as pl
from jax.experimental.pallas import tpu as pltpu
from jax.experimental.pallas import tpu_sc as plsc

_N, _D, _K = 16384, 512, 128
_IDX = np.asarray(jax.random.permutation(jax.random.key(42), _D))[:_K].tolist()

_NW = 32
_ROWS_W = _N // _NW
_CHUNK = 64
_TILES = _CHUNK // 16


def _sc_body(x_hbm, c_hbm, out_hbm, xbuf, cbuf, sem_x, sem_c):
    wid = lax.axis_index("s") * 2 + lax.axis_index("c")
    base = wid * _ROWS_W

    def chunk(ci, carry):
        r0 = base + ci * _CHUNK
        cpx = pltpu.async_copy(x_hbm.at[pl.ds(r0, _CHUNK)], xbuf, sem_x)
        cpc = pltpu.async_copy(c_hbm.at[pl.ds(r0, _CHUNK)], cbuf, sem_c)
        cpx.wait()
        cpc.wait()
        for rt in range(_TILES):
            rows = lax.iota(jnp.int32, 16) + rt * 16
            for j in _IDX:
                cols = jnp.full((16,), j, jnp.int32)
                v = plsc.load_gather(cbuf, [rows, cols])
                plsc.store_scatter(xbuf, [rows, cols], v)
        pltpu.sync_copy(xbuf, out_hbm.at[pl.ds(r0, _CHUNK)])
        return carry

    lax.fori_loop(0, _ROWS_W // _CHUNK, chunk, 0)


_sc_kernel = functools.partial(
    pl.kernel,
    out_type=jax.ShapeDtypeStruct((_N, _D), jnp.float32),
    mesh=plsc.VectorSubcoreMesh(core_axis_name="c", subcore_axis_name="s"),
    compiler_params=pltpu.CompilerParams(
        use_tc_tiling_on_sc=False, needs_layout_passes=False
    ),
    scratch_types=[
        pltpu.VMEM((_CHUNK, _D), jnp.float32),
        pltpu.VMEM((_CHUNK, _D), jnp.float32),
        pltpu.SemaphoreType.DMA,
        pltpu.SemaphoreType.DMA,
    ],
)(_sc_body)


def kernel(x, concepts):
    return _sc_kernel(x, concepts)


# speedup vs baseline: 1.1835x; 1.1835x over previous
import functools

import numpy as np
import jax
import jax.numpy as jnp
from jax import lax
from jax.experimental import pallas as pl
from jax.experimental.pallas import tpu as pltpu
from jax.experimental.pallas import tpu_sc as plsc

_N, _D, _K = 16384, 512, 128
_IDX = np.asarray(jax.random.permutation(jax.random.key(42), _D))[:_K].tolist()
_MASK = np.zeros((_D,), np.int32)
_MASK[_IDX] = 1
_GROUPS = _D // 16
_G_MIXED = [g for g in range(_GROUPS) if _MASK[g * 16:(g + 1) * 16].any()]

_NW = 32
_ROWS_W = _N // _NW
_CHUNK = 32
_NCHUNK = _ROWS_W // _CHUNK
_XBUFS = 4
_CBUFS = 3


def _sc_body(x_hbm, c_hbm, m_hbm, out_hbm, mbuf, sem_m, *bufs):
    xb = bufs[:_XBUFS]
    cb = bufs[_XBUFS:_XBUFS + _CBUFS]
    sin_x = bufs[_XBUFS + _CBUFS:2 * _XBUFS + _CBUFS]
    sout_x = bufs[2 * _XBUFS + _CBUFS:3 * _XBUFS + _CBUFS]
    sin_c = bufs[3 * _XBUFS + _CBUFS:]

    wid = lax.axis_index("s") * 2 + lax.axis_index("c")
    base = wid * _ROWS_W

    pltpu.async_copy(m_hbm, mbuf, sem_m).wait()

    def start_in(g):
        r0 = base + g * _CHUNK
        pltpu.async_copy(x_hbm.at[pl.ds(r0, _CHUNK)], xb[g % _XBUFS],
                         sin_x[g % _XBUFS])
        pltpu.async_copy(c_hbm.at[pl.ds(r0, _CHUNK)], cb[g % _CBUFS],
                         sin_c[g % _CBUFS])

    def wait_in(g):
        pltpu.make_async_copy(x_hbm.at[pl.ds(0, _CHUNK)], xb[g % _XBUFS],
                              sin_x[g % _XBUFS]).wait()
        pltpu.make_async_copy(c_hbm.at[pl.ds(0, _CHUNK)], cb[g % _CBUFS],
                              sin_c[g % _CBUFS]).wait()

    def start_out(g):
        r0 = base + g * _CHUNK
        pltpu.async_copy(xb[g % _XBUFS], out_hbm.at[pl.ds(r0, _CHUNK)],
                         sout_x[g % _XBUFS])

    def wait_out(g):
        pltpu.make_async_copy(xb[g % _XBUFS], out_hbm.at[pl.ds(0, _CHUNK)],
                              sout_x[g % _XBUFS]).wait()

    def compute(g):
        x, c = xb[g % _XBUFS], cb[g % _CBUFS]
        for cg in _G_MIXED:
            mv = mbuf[pl.ds(cg * 16, 16)] != 0

            def row(r, _):
                x[r, pl.ds(cg * 16, 16)] = jnp.where(
                    mv, c[r, pl.ds(cg * 16, 16)], x[r, pl.ds(cg * 16, 16)])
                return 0

            lax.fori_loop(0, _CHUNK, row, 0)

    start_in(0)
    start_in(1)
    for g in range(_NCHUNK):
        if g + 2 < _NCHUNK:
            if g - 2 >= 0:
                wait_out(g - 2)
            start_in(g + 2)
        wait_in(g)
        compute(g)
        start_out(g)
    wait_out(_NCHUNK - 2)
    wait_out(_NCHUNK - 1)


_sc_kernel = functools.partial(
    pl.kernel,
    out_type=jax.ShapeDtypeStruct((_N, _D), jnp.float32),
    mesh=plsc.VectorSubcoreMesh(core_axis_name="c", subcore_axis_name="s"),
    compiler_params=pltpu.CompilerParams(
        use_tc_tiling_on_sc=False, needs_layout_passes=False
    ),
    scratch_types=(
        [pltpu.VMEM((_D,), jnp.int32), pltpu.SemaphoreType.DMA]
        + [pltpu.VMEM((_CHUNK, _D), jnp.float32) for _ in range(_XBUFS)]
        + [pltpu.VMEM((_CHUNK, _D), jnp.float32) for _ in range(_CBUFS)]
        + [pltpu.SemaphoreType.DMA for _ in range(2 * _XBUFS + _CBUFS)]
    ),
)(_sc_body)


def kernel(x, concepts):
    return _sc_kernel(x, concepts, jnp.asarray(_MASK))
